# Initial kernel scaffold; baseline (speedup 1.0000x reference)
import jax
import jax.numpy as jnp
from jax.experimental import pallas as pl


def kernel(x, edge_index, Wl0, bl0, Wr0, g0, be0, Wl1, bl1, Wr1, g1, be1, Wl2, bl2, Wr2):
    raise NotImplementedError("write your pallas kernel here")


# speedup vs baseline: 4.6081x; 4.6081x over previous
import functools

import jax
import jax.numpy as jnp
from jax import lax
from jax.experimental import pallas as pl
from jax.experimental.pallas import tpu as pltpu
from jax.experimental.pallas import tpu_sc as plsc

N = 10000
E = 320000
EPS = 1e-5

NC = 2
NS = 16
NW = NC * NS
CH = 128
CHUNKS = 79
E_PAD = NW * CHUNKS * CH
N_ACC = 10240
ROWS_PER_SUB = N_ACC // NS


def _sc_aggregate(d, with_cnt):
    mesh = plsc.VectorSubcoreMesh(core_axis_name="c", subcore_axis_name="s")
    out_type = [jax.ShapeDtypeStruct((NC, N_ACC, d), jnp.float32)]
    scratch = [
        pltpu.VMEM((CHUNKS, CH), jnp.int32),
        pltpu.VMEM((CHUNKS, CH), jnp.int32),
        pltpu.VMEM((CH, d), jnp.float32),
        pltpu.VMEM_SHARED((N_ACC, d), jnp.float32),
        pltpu.SemaphoreType.DMA,
    ]
    if with_cnt:
        out_type.append(jax.ShapeDtypeStruct((NC, N_ACC), jnp.float32))
        scratch += [
            pltpu.VMEM((CH,), jnp.float32),
            pltpu.VMEM_SHARED((N_ACC,), jnp.float32),
        ]

    def body(p_hbm, src_hbm, dst_hbm, *rest):
        if with_cnt:
            out_hbm, cnt_hbm, src_v, dst_v, rows_v, acc, sem, ones_v, cacc = rest
        else:
            out_hbm, src_v, dst_v, rows_v, acc, sem = rest
        c = lax.axis_index("c")
        s = lax.axis_index("s")
        w = c * NS + s

        zero = jnp.zeros((16,), jnp.float32)

        def zrow(i, _):
            for j in range(d // 16):
                rows_v[i, pl.ds(j * 16, 16)] = zero
            return 0

        lax.fori_loop(0, CH, zrow, 0, unroll=4)
        base = s * ROWS_PER_SUB
        for k in range(ROWS_PER_SUB // CH):
            pltpu.sync_copy(rows_v, acc.at[pl.ds(base + k * CH, CH)])
        if with_cnt:
            for i in range(CH // 16):
                ones_v[pl.ds(i * 16, 16)] = zero
            for k in range(ROWS_PER_SUB // CH):
                pltpu.sync_copy(ones_v, cacc.at[pl.ds(base + k * CH, CH)])
            one = jnp.full((16,), 1.0, jnp.float32)
            for i in range(CH // 16):
                ones_v[pl.ds(i * 16, 16)] = one
        plsc.subcore_barrier()

        pltpu.sync_copy(src_hbm.at[w], src_v)
        pltpu.sync_copy(dst_hbm.at[w], dst_v)

        def chunk(j, _):
            pltpu.async_copy(p_hbm.at[src_v.at[j]], rows_v, sem).wait()
            pltpu.sync_copy(rows_v, acc.at[dst_v.at[j]], add=True)
            if with_cnt:
                pltpu.sync_copy(ones_v, cacc.at[dst_v.at[j]], add=True)
            return 0

        lax.fori_loop(0, CHUNKS, chunk, 0)
        plsc.subcore_barrier()

        for k in range(ROWS_PER_SUB // CH):
            off = base + k * CH
            pltpu.sync_copy(
                acc.at[pl.ds(off, CH)], out_hbm.at[c, pl.ds(off, CH)]
            )
            if with_cnt:
                pltpu.sync_copy(
                    cacc.at[pl.ds(off, CH)], cnt_hbm.at[c, pl.ds(off, CH)]
                )
        return None

    params = None
    if d % 128 != 0:
        params = pltpu.CompilerParams(use_tc_tiling_on_sc=False)
    return pl.kernel(body, out_type=out_type, mesh=mesh, scratch_types=scratch,
                     compiler_params=params)


def _tc_first(x, wl, wr):
    def body(x_ref, wl_ref, wr_ref, p_ref, s_ref):
        xv = x_ref[...]
        p_ref[...] = jnp.dot(xv, wl_ref[...], preferred_element_type=jnp.float32)
        s_ref[...] = jnp.dot(xv, wr_ref[...], preferred_element_type=jnp.float32)

    h = wl.shape[1]
    return pl.pallas_call(
        body,
        out_shape=[
            jax.ShapeDtypeStruct((N, h), jnp.float32),
            jax.ShapeDtypeStruct((N, h), jnp.float32),
        ],
    )(x, wl, wr)


def _tc_mid(aggp, cnt_or_inv, selfp, bl, g, be, wl, wr, first):
    def body(agg_ref, c_ref, s_ref, bl_ref, g_ref, be_ref, wl_ref, wr_ref,
             p_ref, sn_ref, inv_ref):
        if first:
            cnt = c_ref[0, :N] + c_ref[1, :N]
            inv = (1.0 / jnp.maximum(cnt, 1.0))[:, None]
        else:
            inv = c_ref[...]
        z = (agg_ref[0, :N] + agg_ref[1, :N]) * inv + s_ref[...] \
            + bl_ref[...][None, :]
        mu = jnp.mean(z, axis=0)
        zc = z - mu[None, :]
        var = jnp.mean(zc * zc, axis=0)
        hsc = zc * lax.rsqrt(var + EPS) * g_ref[...][None, :] + be_ref[...][None, :]
        hv = jnp.maximum(hsc, 0.0)
        p_ref[...] = jnp.dot(hv, wl_ref[...], preferred_element_type=jnp.float32)
        sn_ref[...] = jnp.dot(hv, wr_ref[...], preferred_element_type=jnp.float32)
        inv_ref[...] = inv

    h = wl.shape[1]
    return pl.pallas_call(
        body,
        out_shape=[
            jax.ShapeDtypeStruct((N, h), jnp.float32),
            jax.ShapeDtypeStruct((N, h), jnp.float32),
            jax.ShapeDtypeStruct((N, 1), jnp.float32),
        ],
    )(aggp, cnt_or_inv, selfp, bl, g, be, wl, wr)


def _tc_last(aggp, inv, selfp, bl):
    def body(agg_ref, inv_ref, s_ref, bl_ref, o_ref):
        z = (agg_ref[0, :N] + agg_ref[1, :N]) * inv_ref[...] \
            + s_ref[...] + bl_ref[...][None, :]
        m = jnp.max(z, axis=-1, keepdims=True)
        zs = z - m
        lse = jnp.log(jnp.sum(jnp.exp(zs), axis=-1, keepdims=True))
        o_ref[...] = zs - lse

    d = selfp.shape[1]
    return pl.pallas_call(
        body,
        out_shape=jax.ShapeDtypeStruct((N, d), jnp.float32),
    )(aggp, inv, selfp, bl)


def kernel(x, edge_index, Wl0, bl0, Wr0, g0, be0, Wl1, bl1, Wr1, g1, be1,
           Wl2, bl2, Wr2):
    src = edge_index[0]
    dst = edge_index[1]
    pad = E_PAD - E
    src3 = jnp.concatenate([src, jnp.zeros((pad,), jnp.int32)]).reshape(
        NW, CHUNKS, CH)
    dst3 = jnp.concatenate([dst, jnp.full((pad,), N, jnp.int32)]).reshape(
        NW, CHUNKS, CH)

    p0, s0 = _tc_first(x, Wl0, Wr0)
    agg0, cnt = _sc_aggregate(128, True)(p0, src3, dst3)
    p1, s1, inv = _tc_mid(agg0, cnt, s0, bl0, g0, be0, Wl1, Wr1, True)
    agg1, = _sc_aggregate(128, False)(p1, src3, dst3)
    p2, s2, inv = _tc_mid(agg1, inv, s1, bl1, g1, be1, Wl2, Wr2, False)
    agg2, = _sc_aggregate(64, False)(p2, src3, dst3)
    return _tc_last(agg2, inv, s2, bl2)
